# Initial kernel scaffold; baseline (speedup 1.0000x reference)
import jax
import jax.numpy as jnp
from jax.experimental import pallas as pl


def kernel(queries, keys, values):
    raise NotImplementedError("write your pallas kernel here")


# speedup vs baseline: 1.2427x; 1.2427x over previous
import functools
import math

import jax
import jax.numpy as jnp
import numpy as np
from jax.experimental import pallas as pl
from jax.experimental.pallas import tpu as pltpu


@functools.lru_cache(maxsize=None)
def _sample_counts(L_Q: int, L_K: int, U_part: int):
    with jax.ensure_compile_time_eval():
        idx = jax.random.randint(jax.random.key(42), (L_Q, U_part), 0, L_K)
    idx_np = np.asarray(idx, dtype=np.int64)
    cnt = np.zeros((L_Q, L_K), dtype=np.int8)
    np.add.at(cnt, (np.arange(L_Q)[:, None], idx_np), 1)
    return cnt


def _prob_attn_kernel(q_ref, k_ref, v_ref, cnt_ref, out_ref, attn_ref, *,
                      L_K: int, u: int, bq: int, hp: int, D: int,
                      scale: float):
    L_Q = q_ref.shape[0]
    iota = jax.lax.broadcasted_iota(jnp.int32, (L_Q, 1), 0)
    for h in range(hp):
        lanes = slice(h * D, (h + 1) * D)
        q = q_ref[:, lanes]
        k = k_ref[:, lanes]

        m_cols = []
        for i in range(L_Q // bq):
            qb = q[i * bq:(i + 1) * bq, :]
            s = jax.lax.dot_general(qb, k, (((1,), (1,)), ((), ())),
                                    preferred_element_type=jnp.float32)
            c = cnt_ref[i * bq:(i + 1) * bq, :].astype(jnp.float32)
            rmax = jnp.max(jnp.where(c > 0.0, s, -jnp.inf), axis=1,
                           keepdims=True)
            rsum = jnp.sum(s * c, axis=1, keepdims=True)
            m_cols.append(rmax - rsum * (1.0 / L_K))
        m = jnp.concatenate(m_cols, axis=0)

        idxs = []
        cur = m
        for _ in range(u):
            mval = jnp.max(cur)
            j = jnp.min(jnp.where(cur == mval, iota, jnp.int32(2 ** 30)))
            idxs.append(j)
            cur = jnp.where(iota == j, -jnp.inf, cur)

        q_rows = [q_ref[pl.ds(j, 1), lanes] for j in idxs]
        qr = jnp.concatenate(q_rows, axis=0)
        scores = jax.lax.dot_general(qr, k, (((1,), (1,)), ((), ())),
                                     preferred_element_type=jnp.float32)
        scores = scores * scale
        smax = jnp.max(scores, axis=1, keepdims=True)
        e = jnp.exp(scores - smax)
        attn = e / jnp.sum(e, axis=1, keepdims=True)
        attn_ref[h] = attn

        v = v_ref[:, lanes]
        upd = jax.lax.dot_general(attn, v, (((1,), (0,)), ((), ())),
                                  preferred_element_type=jnp.float32)

        vmean = jnp.sum(v, axis=0, keepdims=True) * (1.0 / L_K)
        out_ref[:, lanes] = jnp.broadcast_to(vmean, (L_Q, D))
        for s_i, j in enumerate(idxs):
            out_ref[pl.ds(j, 1), lanes] = upd[s_i:s_i + 1, :]


def kernel(queries, keys, values):
    B, L_Q, H, D = queries.shape
    _, L_K, _, _ = keys.shape
    factor = 1
    U_part = factor * int(np.ceil(np.log(L_K)))
    u = factor * int(np.ceil(np.log(L_Q)))
    U_part = min(U_part, L_K)
    u = min(u, L_Q)
    scale = 1.0 / math.sqrt(D)
    bq = min(256, L_Q)
    hp = 2 if (D == 64 and H % 2 == 0) else 1

    cnt = jnp.asarray(_sample_counts(L_Q, L_K, U_part))
    qf = queries.reshape(B, L_Q, H * D)
    kf = keys.reshape(B, L_K, H * D)
    vf = values.reshape(B, L_K, H * D)

    n_hb = H // hp
    grid = (B * n_hb,)
    bh_map = lambda i: (i // n_hb, 0, i % n_hb)

    out, attn = pl.pallas_call(
        functools.partial(_prob_attn_kernel, L_K=L_K, u=u, bq=bq, hp=hp, D=D,
                          scale=scale),
        grid=grid,
        in_specs=[
            pl.BlockSpec((None, L_Q, hp * D), bh_map),
            pl.BlockSpec((None, L_K, hp * D), bh_map),
            pl.BlockSpec((None, L_K, hp * D), bh_map),
            pl.BlockSpec((L_Q, L_K), lambda i: (0, 0)),
        ],
        out_specs=[
            pl.BlockSpec((None, L_Q, hp * D), bh_map),
            pl.BlockSpec((None, hp, u, L_K),
                         lambda i: (i // n_hb, i % n_hb, 0, 0)),
        ],
        out_shape=[
            jax.ShapeDtypeStruct((B, L_Q, H * D), jnp.float32),
            jax.ShapeDtypeStruct((B, H, u, L_K), jnp.float32),
        ],
        compiler_params=pltpu.CompilerParams(
            dimension_semantics=("arbitrary",),
        ),
    )(qf, kf, vf, cnt)
    return (out.reshape(B, L_Q, H, D), attn)
